# baseline (speedup 1.0000x reference)
import functools
import math

import jax
import jax.numpy as jnp
from jax.experimental import pallas as pl
from jax.experimental.pallas import tpu as pltpu

_GAMMA = 3.0
_ALPHA = 1.5
_GRID = 32
_EPS = 1e-7

_L = 2048
_RB = 8
_BLK = _L * _RB


_ATAN_C = (1.0, -0.3333312, 0.19993663, -0.14212675, 0.1067899,
           -0.07590766, 0.04377373, -0.01677049, 0.00303406)


def _atan_pos(r):
    inv = 1.0 / r
    t = jnp.minimum(r, inv)
    z = t * t
    p = jnp.full_like(z, _ATAN_C[-1])
    for c in _ATAN_C[-2::-1]:
        p = p * z + c
    at = t * p
    return jnp.where(r > 1.0, (jnp.pi / 2) - at, at)


def _ciou_base(px, py, pw, ph, tx, ty, tw, th):
    hw1, hh1 = pw * 0.5, ph * 0.5
    hw2, hh2 = tw * 0.5, th * 0.5
    b1x1, b1x2 = px - hw1, px + hw1
    b1y1, b1y2 = py - hh1, py + hh1
    b2x1, b2x2 = tx - hw2, tx + hw2
    b2y1, b2y2 = ty - hh2, ty + hh2
    iw = jnp.maximum(jnp.minimum(b1x2, b2x2) - jnp.maximum(b1x1, b2x1), 0.0)
    ih = jnp.maximum(jnp.minimum(b1y2, b2y2) - jnp.maximum(b1y1, b2y1), 0.0)
    inter = iw * ih
    union = pw * ph + tw * th - inter + _EPS
    iou = inter / union
    cw = jnp.maximum(b1x2, b2x2) - jnp.minimum(b1x1, b2x1)
    ch = jnp.maximum(b1y2, b2y2) - jnp.minimum(b1y1, b2y1)
    c2 = cw * cw + ch * ch + _EPS
    dx = b2x1 + b2x2 - b1x1 - b1x2
    dy = b2y1 + b2y2 - b1y1 - b1y2
    rho2 = (dx * dx + dy * dy) * 0.25
    v = (4.0 / (jnp.pi ** 2)) * (_atan_pos(tw / th) - _atan_pos(pw / ph)) ** 2
    a = v / (v - iou + (1.0 + _EPS))
    ciou = iou - (rho2 / c2 + v * a)
    one_m = 1.0 - ciou
    base = one_m * one_m * one_m
    return base / (tw * th + 1e-7)


def _body(p_ref, t_ref, hist_o, base_o):
    px, py, pw, ph = p_ref[0], p_ref[1], p_ref[2], p_ref[3]
    tx, ty, tw, th = t_ref[0], t_ref[1], t_ref[2], t_ref[3]
    base = _ciou_base(px, py, pw, ph, tx, ty, tw, th)
    base_o[0, 0, :] = jnp.sum(base, axis=0)

    gx = jnp.clip((tx * _GRID).astype(jnp.int32), 0, _GRID - 1)
    gy = jnp.clip((ty * _GRID).astype(jnp.int32), 0, _GRID - 1)
    iota = jax.lax.broadcasted_iota(jnp.int32, (_GRID, _L), 0)
    acc = jnp.zeros((_GRID, _GRID), jnp.float32)
    for r in range(_RB):
        yr = jnp.broadcast_to(gy[r:r + 1, :], (_GRID, _L))
        xr = jnp.broadcast_to(gx[r:r + 1, :], (_GRID, _L))
        ohy = jnp.where(yr == iota, 1.0, 0.0)
        ohx = jnp.where(xr == iota, 1.0, 0.0)
        acc = acc + jax.lax.dot_general(
            ohy, ohx, (((1,), (1,)), ((), ())),
            preferred_element_type=jnp.float32)
    hist_o[0] = acc


@jax.jit
def kernel(pred_boxes, target_boxes):
    n = pred_boxes.shape[0]
    g = math.ceil(n / _BLK)
    npad = g * _BLK
    p = npad - n

    def comp(b):
        c = jnp.pad(b, ((0, p), (0, 0)), constant_values=0.5)
        return c.T.reshape(4, g * _RB, _L)

    comps = [comp(pred_boxes), comp(target_boxes)]

    hist_parts, base_parts = pl.pallas_call(
        _body,
        grid=(g,),
        in_specs=[pl.BlockSpec((4, _RB, _L), lambda gi: (0, gi, 0))] * 2,
        out_specs=[
            pl.BlockSpec((1, _GRID, _GRID), lambda gi: (gi, 0, 0)),
            pl.BlockSpec((1, 1, _L), lambda gi: (gi, 0, 0)),
        ],
        out_shape=[
            jax.ShapeDtypeStruct((g, _GRID, _GRID), jnp.float32),
            jax.ShapeDtypeStruct((g, 1, _L), jnp.float32),
        ],
        compiler_params=pltpu.CompilerParams(
            dimension_semantics=("parallel",)),
    )(*comps)

    counts = jnp.sum(hist_parts, axis=0)
    counts = counts.at[_GRID // 2, _GRID // 2].add(-float(p))
    density = counts / jnp.max(counts)
    mean_dw = jnp.mean(1.0 + _ALPHA * density)
    mean_base = jnp.sum(base_parts) / n
    return mean_base * mean_dw


# speedup vs baseline: 1.1810x; 1.1810x over previous
import functools
import math

import jax
import jax.numpy as jnp
from jax.experimental import pallas as pl
from jax.experimental.pallas import tpu as pltpu

_GAMMA = 3.0
_ALPHA = 1.5
_GRID = 32
_EPS = 1e-7

_L = 2048
_RB = 16
_BLK = _L * _RB


_ATAN_C = (1.0, -0.3333312, 0.19993663, -0.14212675, 0.1067899,
           -0.07590766, 0.04377373, -0.01677049, 0.00303406)


def _atan_pos(r):
    inv = 1.0 / r
    t = jnp.minimum(r, inv)
    z = t * t
    p = jnp.full_like(z, _ATAN_C[-1])
    for c in _ATAN_C[-2::-1]:
        p = p * z + c
    at = t * p
    return jnp.where(r > 1.0, (jnp.pi / 2) - at, at)


def _atan_diff_sq(a, b):
    q = jnp.abs(a - b) / (1.0 + a * b)
    at = _atan_pos(q)
    return at * at


def _ciou_base(px, py, pw, ph, tx, ty, tw, th):
    hw1, hh1 = pw * 0.5, ph * 0.5
    hw2, hh2 = tw * 0.5, th * 0.5
    b1x1, b1x2 = px - hw1, px + hw1
    b1y1, b1y2 = py - hh1, py + hh1
    b2x1, b2x2 = tx - hw2, tx + hw2
    b2y1, b2y2 = ty - hh2, ty + hh2
    iw = jnp.maximum(jnp.minimum(b1x2, b2x2) - jnp.maximum(b1x1, b2x1), 0.0)
    ih = jnp.maximum(jnp.minimum(b1y2, b2y2) - jnp.maximum(b1y1, b2y1), 0.0)
    inter = iw * ih
    union = pw * ph + tw * th - inter + _EPS
    iou = inter / union
    cw = jnp.maximum(b1x2, b2x2) - jnp.minimum(b1x1, b2x1)
    ch = jnp.maximum(b1y2, b2y2) - jnp.minimum(b1y1, b2y1)
    c2 = cw * cw + ch * ch + _EPS
    dx = b2x1 + b2x2 - b1x1 - b1x2
    dy = b2y1 + b2y2 - b1y1 - b1y2
    rho2 = (dx * dx + dy * dy) * 0.25
    v = (4.0 / (jnp.pi ** 2)) * _atan_diff_sq(
        tw / jnp.maximum(th, 1e-20), pw / jnp.maximum(ph, 1e-20))
    a = v / (v - iou + (1.0 + _EPS))
    ciou = iou - (rho2 / c2 + v * a)
    one_m = 1.0 - ciou
    base = one_m * one_m * one_m
    return base / (tw * th + 1e-7)


def _body(p_ref, t_ref, hist_o, base_o):
    px, py, pw, ph = p_ref[0], p_ref[1], p_ref[2], p_ref[3]
    tx, ty, tw, th = t_ref[0], t_ref[1], t_ref[2], t_ref[3]
    base = _ciou_base(px, py, pw, ph, tx, ty, tw, th)
    base_o[0, 0, :] = jnp.sum(base, axis=0)

    gx = jnp.clip((tx * _GRID).astype(jnp.int32), 0, _GRID - 1)
    gy = jnp.clip((ty * _GRID).astype(jnp.int32), 0, _GRID - 1)
    iota = jax.lax.broadcasted_iota(jnp.int32, (_GRID, _L), 0)
    acc = jnp.zeros((_GRID, _GRID), jnp.float32)
    for r in range(_RB):
        yr = jnp.broadcast_to(gy[r:r + 1, :], (_GRID, _L))
        xr = jnp.broadcast_to(gx[r:r + 1, :], (_GRID, _L))
        ohy = jnp.where(yr == iota, 1.0, 0.0)
        ohx = jnp.where(xr == iota, 1.0, 0.0)
        acc = acc + jax.lax.dot_general(
            ohy, ohx, (((1,), (1,)), ((), ())),
            preferred_element_type=jnp.float32)
    hist_o[0] = acc


@jax.jit
def kernel(pred_boxes, target_boxes):
    n = pred_boxes.shape[0]
    g = math.ceil(n / _BLK)
    npad = g * _BLK
    p = npad - n

    def comp(b):
        c = jnp.pad(b, ((0, p), (0, 0)), constant_values=0.5)
        return c.T.reshape(4, g * _RB, _L)

    comps = [comp(pred_boxes), comp(target_boxes)]

    hist_parts, base_parts = pl.pallas_call(
        _body,
        grid=(g,),
        in_specs=[pl.BlockSpec((4, _RB, _L), lambda gi: (0, gi, 0))] * 2,
        out_specs=[
            pl.BlockSpec((1, _GRID, _GRID), lambda gi: (gi, 0, 0)),
            pl.BlockSpec((1, 1, _L), lambda gi: (gi, 0, 0)),
        ],
        out_shape=[
            jax.ShapeDtypeStruct((g, _GRID, _GRID), jnp.float32),
            jax.ShapeDtypeStruct((g, 1, _L), jnp.float32),
        ],
        compiler_params=pltpu.CompilerParams(
            dimension_semantics=("parallel",)),
    )(*comps)

    counts = jnp.sum(hist_parts, axis=0)
    counts = counts.at[_GRID // 2, _GRID // 2].add(-float(p))
    density = counts / jnp.max(counts)
    mean_dw = jnp.mean(1.0 + _ALPHA * density)
    mean_base = jnp.sum(base_parts) / n
    return mean_base * mean_dw


# speedup vs baseline: 1.1817x; 1.0006x over previous
import functools
import math

import jax
import jax.numpy as jnp
from jax.experimental import pallas as pl
from jax.experimental.pallas import tpu as pltpu

_GAMMA = 3.0
_ALPHA = 1.5
_GRID = 32
_EPS = 1e-7

_L = 2048
_RB = 16
_BLK = _L * _RB


_ATAN_C = (1.0, -0.3333312, 0.19993663, -0.14212675, 0.1067899,
           -0.07590766, 0.04377373, -0.01677049, 0.00303406)


def _atan_pos(r):
    inv = 1.0 / r
    t = jnp.minimum(r, inv)
    z = t * t
    p = jnp.full_like(z, _ATAN_C[-1])
    for c in _ATAN_C[-2::-1]:
        p = p * z + c
    at = t * p
    return jnp.where(r > 1.0, (jnp.pi / 2) - at, at)


def _atan_diff_sq(a, b):
    q = jnp.abs(a - b) / (1.0 + a * b)
    at = _atan_pos(q)
    return at * at


def _ciou_base(px, py, pw, ph, tx, ty, tw, th):
    hw1, hh1 = pw * 0.5, ph * 0.5
    hw2, hh2 = tw * 0.5, th * 0.5
    b1x1, b1x2 = px - hw1, px + hw1
    b1y1, b1y2 = py - hh1, py + hh1
    b2x1, b2x2 = tx - hw2, tx + hw2
    b2y1, b2y2 = ty - hh2, ty + hh2
    iw = jnp.maximum(jnp.minimum(b1x2, b2x2) - jnp.maximum(b1x1, b2x1), 0.0)
    ih = jnp.maximum(jnp.minimum(b1y2, b2y2) - jnp.maximum(b1y1, b2y1), 0.0)
    inter = iw * ih
    union = pw * ph + tw * th - inter + _EPS
    iou = inter / union
    cw = jnp.maximum(b1x2, b2x2) - jnp.minimum(b1x1, b2x1)
    ch = jnp.maximum(b1y2, b2y2) - jnp.minimum(b1y1, b2y1)
    c2 = cw * cw + ch * ch + _EPS
    dx = b2x1 + b2x2 - b1x1 - b1x2
    dy = b2y1 + b2y2 - b1y1 - b1y2
    rho2 = (dx * dx + dy * dy) * 0.25
    v = (4.0 / (jnp.pi ** 2)) * _atan_diff_sq(
        tw / jnp.maximum(th, 1e-20), pw / jnp.maximum(ph, 1e-20))
    a = v / (v - iou + (1.0 + _EPS))
    ciou = iou - (rho2 / c2 + v * a)
    one_m = 1.0 - ciou
    base = one_m * one_m * one_m
    return base / (tw * th + 1e-7)


_CH = 8


def _body(p_ref, t_ref, hist_o, base_o):
    acc_row = jnp.zeros((1, _L), jnp.float32)
    for r0 in range(0, _RB, _CH):
        r1 = r0 + _CH
        base = _ciou_base(
            p_ref[0, r0:r1, :], p_ref[1, r0:r1, :],
            p_ref[2, r0:r1, :], p_ref[3, r0:r1, :],
            t_ref[0, r0:r1, :], t_ref[1, r0:r1, :],
            t_ref[2, r0:r1, :], t_ref[3, r0:r1, :])
        acc_row = acc_row + jnp.sum(base, axis=0, keepdims=True)
    base_o[0] = acc_row

    tx = t_ref[0]
    ty = t_ref[1]
    gx = jnp.clip((tx * _GRID).astype(jnp.int32), 0, _GRID - 1)
    gy = jnp.clip((ty * _GRID).astype(jnp.int32), 0, _GRID - 1)
    iota = jax.lax.broadcasted_iota(jnp.int32, (_GRID, _L), 0)
    acc = jnp.zeros((_GRID, _GRID), jnp.float32)
    for r in range(_RB):
        yr = jnp.broadcast_to(gy[r:r + 1, :], (_GRID, _L))
        xr = jnp.broadcast_to(gx[r:r + 1, :], (_GRID, _L))
        ohy = jnp.where(yr == iota, 1.0, 0.0)
        ohx = jnp.where(xr == iota, 1.0, 0.0)
        acc = acc + jax.lax.dot_general(
            ohy, ohx, (((1,), (1,)), ((), ())),
            preferred_element_type=jnp.float32)
    hist_o[0] = acc


@jax.jit
def kernel(pred_boxes, target_boxes):
    n = pred_boxes.shape[0]
    g = math.ceil(n / _BLK)
    npad = g * _BLK
    p = npad - n

    def comp(b):
        c = jnp.pad(b, ((0, p), (0, 0)), constant_values=0.5)
        return c.T.reshape(4, g * _RB, _L)

    comps = [comp(pred_boxes), comp(target_boxes)]

    hist_parts, base_parts = pl.pallas_call(
        _body,
        grid=(g,),
        in_specs=[pl.BlockSpec((4, _RB, _L), lambda gi: (0, gi, 0))] * 2,
        out_specs=[
            pl.BlockSpec((1, _GRID, _GRID), lambda gi: (gi, 0, 0)),
            pl.BlockSpec((1, 1, _L), lambda gi: (gi, 0, 0)),
        ],
        out_shape=[
            jax.ShapeDtypeStruct((g, _GRID, _GRID), jnp.float32),
            jax.ShapeDtypeStruct((g, 1, _L), jnp.float32),
        ],
        compiler_params=pltpu.CompilerParams(
            dimension_semantics=("parallel",)),
    )(*comps)

    counts = jnp.sum(hist_parts, axis=0)
    counts = counts.at[_GRID // 2, _GRID // 2].add(-float(p))
    density = counts / jnp.max(counts)
    mean_dw = jnp.mean(1.0 + _ALPHA * density)
    mean_base = jnp.sum(base_parts) / n
    return mean_base * mean_dw
